# baseline (speedup 1.0000x reference)
import jax
import jax.numpy as jnp
import numpy as np
from jax import lax
from jax.experimental import pallas as pl
from jax.experimental.pallas import tpu as pltpu
from jax.experimental.pallas import tpu_sc as plsc

N = 10000
E = 320000
H = 16
C = 128
HC = 2048
HHC = HC // 2
NCLASS = 16
NGRAPH = 64

NW = 32
EPW = E // NW
CH = 16
NCHUNK = EPW // CH
CHB = 80
NCHB = EPW // CHB
CHC = 8
NCHC = EPW // CHC
NBLK = 5
CBLK = NCHC // NBLK
NGRP = E // 16

F32 = jnp.float32
I32 = jnp.int32
BF16 = jnp.bfloat16

_SC_PARAMS = dict(
    mesh=plsc.VectorSubcoreMesh(core_axis_name="c", subcore_axis_name="s"),
    compiler_params=pltpu.CompilerParams(
        use_tc_tiling_on_sc=False, needs_layout_passes=False),
)


def _worker():
    cid = lax.axis_index("c")
    sid = lax.axis_index("s")
    return cid, sid, sid * 2 + cid


def _phase_a_body(xla, xlb, xra, xrb, att_hbm, srcg, dstg, lg_hbm, pm_hbm,
                  pb0, pb1, qb0, qb1, attv, lgbuf, maxbuf, sallv, dallv,
                  sem0, sem1):
    _, _, wid = _worker()
    gbase = wid * NCHUNK
    pltpu.sync_copy(att_hbm, attv)
    pltpu.sync_copy(srcg.at[pl.ds(gbase, NCHUNK)], sallv)
    pltpu.sync_copy(dstg.at[pl.ds(gbase, NCHUNK)], dallv)
    row_iota = lax.iota(I32, 16)
    zero16 = jnp.zeros((16,), F32)
    for h in range(H):
        maxbuf[h, :] = jnp.full((16,), -jnp.inf, F32)

    pbufs, qbufs, sems = (pb0, pb1), (qb0, qb1), (sem0, sem1)
    xlh, xrh = (xla, xlb), (xra, xrb)

    def issue(ci, k, b):
        pltpu.async_copy(xlh[k].at[sallv.at[ci]], pbufs[b], sems[b])
        pltpu.async_copy(xrh[k].at[dallv.at[ci]], qbufs[b], sems[b])

    def wait(b):
        pltpu.make_async_copy(xla.at[sallv.at[0]], pbufs[b], sems[b]).wait()
        pltpu.make_async_copy(xra.at[dallv.at[0]], qbufs[b], sems[b]).wait()

    def compute_half(b, k):
        pbuf, qbuf = pbufs[b], qbufs[b]
        ilv = plsc.PackFormat.INTERLEAVED

        @pl.loop(0, CH, init_carry=tuple(zero16 for _ in range(8)))
        def _edges(e, lgvs):
            oh = jnp.where(row_iota == e, 1.0, 0.0).astype(F32)
            new = []
            for hl in range(8):
                acc = zero16
                for g in range(4):
                    off = hl * 128 + g * 32
                    p0, p1 = plsc.unpack(pbuf[e, pl.ds(off, 32)], format=ilv)
                    q0, q1 = plsc.unpack(qbuf[e, pl.ds(off, 32)], format=ilv)
                    z0 = p0 + q0
                    z1 = p1 + q1
                    a0 = attv[pl.ds(k * HHC + off, 16)]
                    a1 = attv[pl.ds(k * HHC + off + 16, 16)]
                    acc = (acc + jnp.maximum(z0, 0.2 * z0) * a0
                           + jnp.maximum(z1, 0.2 * z1) * a1)
                new.append(lgvs[hl] + jnp.sum(acc) * oh)
            return tuple(new)
        return _edges

    issue(0, 0, 0)

    @pl.loop(0, NCHUNK)
    def _chunk(i):
        issue(i, 1, 1)
        wait(0)
        lo = compute_half(0, 0)

        @pl.when(i + 1 < NCHUNK)
        def _():
            issue(i + 1, 0, 0)
        wait(1)
        hi = compute_half(1, 1)
        for hl in range(8):
            lgbuf[hl, :] = lo[hl]
            maxbuf[hl, :] = jnp.maximum(maxbuf[hl, :], lo[hl])
            lgbuf[8 + hl, :] = hi[hl]
            maxbuf[8 + hl, :] = jnp.maximum(maxbuf[8 + hl, :], hi[hl])
        pltpu.sync_copy(lgbuf, lg_hbm.at[gbase + i])

    pltpu.sync_copy(maxbuf, pm_hbm.at[wid])


def _phase_a():
    return pl.kernel(
        _phase_a_body,
        out_type=[jax.ShapeDtypeStruct((NGRP, 16, 16), F32),
                  jax.ShapeDtypeStruct((NW, 16, 16), F32)],
        scratch_types=[
            pltpu.VMEM((CH, HHC), BF16),
            pltpu.VMEM((CH, HHC), BF16),
            pltpu.VMEM((CH, HHC), BF16),
            pltpu.VMEM((CH, HHC), BF16),
            pltpu.VMEM((HC,), F32),
            pltpu.VMEM((16, 16), F32),
            pltpu.VMEM((16, 16), F32),
            pltpu.VMEM((NCHUNK, CH), I32),
            pltpu.VMEM((NCHUNK, CH), I32),
            pltpu.SemaphoreType.DMA,
            pltpu.SemaphoreType.DMA,
        ],
        **_SC_PARAMS,
    )


def _phase_b_body(lg_hbm, pm_hbm, dst_hbm, denz_hbm, ex_hbm, den0_hbm, den1_hbm,
                  pmall, gsplat, lgc, tmph, exbuf, didx, densh):
    cid, sid, wid = _worker()
    ebase = wid * EPW
    row_iota = lax.iota(I32, 16)

    @pl.when(sid == 0)
    def _():
        pltpu.sync_copy(denz_hbm, densh)
    plsc.subcore_barrier()

    pltpu.sync_copy(pm_hbm, pmall)
    for h in range(H):
        gv = pmall[0, h, :]
        for w in range(1, NW):
            gv = jnp.maximum(gv, pmall[w, h, :])
        gsplat[h, :] = jnp.full((16,), jnp.max(gv), F32)

    @pl.loop(0, NCHB)
    def _chunk(i):
        base = ebase + i * CHB
        g0 = ebase // 16 + i * (CHB // 16)
        pltpu.sync_copy(lg_hbm.at[pl.ds(g0, CHB // 16)], lgc)
        pltpu.sync_copy(dst_hbm.at[pl.ds(base, CHB)], didx)
        for g in range(CHB // 16):
            for h in range(H):
                tmph[h, :] = jnp.exp(lgc[g, h, :] - gsplat[h, :])
            for e in range(16):
                esplat = jnp.full((16,), e, I32)
                exbuf[g * 16 + e, :] = plsc.load_gather(tmph, [row_iota, esplat])
        pltpu.sync_copy(exbuf, ex_hbm.at[pl.ds(base, CHB)])
        pltpu.sync_copy(exbuf, densh.at[didx], add=True)

    plsc.subcore_barrier()

    @pl.when(jnp.logical_and(sid == 0, cid == 0))
    def _():
        pltpu.sync_copy(densh, den0_hbm)

    @pl.when(jnp.logical_and(sid == 0, cid == 1))
    def _():
        pltpu.sync_copy(densh, den1_hbm)


def _phase_b():
    return pl.kernel(
        _phase_b_body,
        out_type=[jax.ShapeDtypeStruct((E, 16), F32),
                  jax.ShapeDtypeStruct((N, 16), F32),
                  jax.ShapeDtypeStruct((N, 16), F32)],
        scratch_types=[
            pltpu.VMEM((NW, 16, 16), F32),
            pltpu.VMEM((16, 16), F32),
            pltpu.VMEM((CHB // 16, 16, 16), F32),
            pltpu.VMEM((16, 16), F32),
            pltpu.VMEM((CHB, 16), F32),
            pltpu.VMEM((CHB,), I32),
            pltpu.VMEM_SHARED((N, 16), F32),
        ],
        **_SC_PARAMS,
    )


def _phase_c_body(xla, xlb, ex_hbm, den0_hbm, den1_hbm, srcc, dstc,
                  outz_hbm, out0_hbm, out1_hbm,
                  xba0, xba1, xbb0, xbb1, exb0, exb1, d0b0, d0b1, d1b0, d1b1,
                  abuf, wbuf, sallv, dallv, sem0, sem1, outsh):
    cid, sid, wid = _worker()
    ebase = wid * EPW

    @pl.when(sid == 0)
    def _():
        pltpu.sync_copy(outz_hbm, outsh)
    plsc.subcore_barrier()

    xbas, xbbs = (xba0, xba1), (xbb0, xbb1)
    exbs, d0bs, d1bs = (exb0, exb1), (d0b0, d0b1), (d1b0, d1b1)
    sems = (sem0, sem1)

    def issue(blk, ci, b):
        base = ebase + (blk * CBLK + ci) * CHC
        pltpu.async_copy(xla.at[sallv.at[ci]], xbas[b], sems[b])
        pltpu.async_copy(xlb.at[sallv.at[ci]], xbbs[b], sems[b])
        pltpu.async_copy(ex_hbm.at[pl.ds(base, CHC)], exbs[b], sems[b])
        pltpu.async_copy(den0_hbm.at[dallv.at[ci]], d0bs[b], sems[b])
        pltpu.async_copy(den1_hbm.at[dallv.at[ci]], d1bs[b], sems[b])

    def wait(b):
        pltpu.make_async_copy(xla.at[sallv.at[0]], xbas[b], sems[b]).wait()
        pltpu.make_async_copy(xlb.at[sallv.at[0]], xbbs[b], sems[b]).wait()
        pltpu.make_async_copy(ex_hbm.at[pl.ds(ebase, CHC)], exbs[b], sems[b]).wait()
        pltpu.make_async_copy(den0_hbm.at[dallv.at[0]], d0bs[b], sems[b]).wait()
        pltpu.make_async_copy(den1_hbm.at[dallv.at[0]], d1bs[b], sems[b]).wait()

    def compute(ci, b):
        xba, xbb, exb, d0b, d1b = xbas[b], xbbs[b], exbs[b], d0bs[b], d1bs[b]
        ilv = plsc.PackFormat.INTERLEAVED
        for e in range(CHC):
            d = (d0b[e, :] + d1b[e, :]) * float(H)
            abuf[e, :] = exb[e, :] / d
        for e in range(CHC):
            zeros8 = tuple(jnp.zeros((16,), F32) for _ in range(8))
            esplat = jnp.full((16,), e, I32)

            @pl.loop(0, 8, init_carry=zeros8)
            def _lo(hh, accs):
                alv = plsc.load_gather(abuf, [esplat, jnp.full((16,), hh, I32)])
                out = list(accs)
                for g in range(4):
                    x0, x1 = plsc.unpack(
                        xba[e, pl.ds(hh * 128 + g * 32, 32)], format=ilv)
                    out[2 * g] = out[2 * g] + alv * x0
                    out[2 * g + 1] = out[2 * g + 1] + alv * x1
                return tuple(out)

            @pl.loop(8, 16, init_carry=_lo)
            def _hi(hh, accs):
                alv = plsc.load_gather(abuf, [esplat, jnp.full((16,), hh, I32)])
                out = list(accs)
                for g in range(4):
                    x0, x1 = plsc.unpack(
                        xbb[e, pl.ds((hh - 8) * 128 + g * 32, 32)], format=ilv)
                    out[2 * g] = out[2 * g] + alv * x0
                    out[2 * g + 1] = out[2 * g + 1] + alv * x1
                return tuple(out)

            for g in range(4):
                wbuf[e, pl.ds(g * 32, 16)] = _hi[2 * g]
                wbuf[e, pl.ds(g * 32 + 16, 16)] = _hi[2 * g + 1]
        pltpu.sync_copy(wbuf, outsh.at[dallv.at[ci]], add=True)

    for blk in range(NBLK):
        cb0 = wid * NCHC + blk * CBLK
        pltpu.sync_copy(srcc.at[pl.ds(cb0, CBLK)], sallv)
        pltpu.sync_copy(dstc.at[pl.ds(cb0, CBLK)], dallv)
        issue(blk, 0, 0)

        @pl.loop(0, CBLK // 2)
        def _pair(p):
            i0 = p * 2
            issue(blk, i0 + 1, 1)
            wait(0)
            compute(i0, 0)

            @pl.when(i0 + 2 < CBLK)
            def _():
                issue(blk, i0 + 2, 0)
            wait(1)
            compute(i0 + 1, 1)

    plsc.subcore_barrier()

    @pl.when(jnp.logical_and(sid == 0, cid == 0))
    def _():
        pltpu.sync_copy(outsh, out0_hbm)

    @pl.when(jnp.logical_and(sid == 0, cid == 1))
    def _():
        pltpu.sync_copy(outsh, out1_hbm)


def _phase_c():
    return pl.kernel(
        _phase_c_body,
        out_type=[jax.ShapeDtypeStruct((N, C), F32),
                  jax.ShapeDtypeStruct((N, C), F32)],
        scratch_types=[
            pltpu.VMEM((CHC, HHC), BF16),
            pltpu.VMEM((CHC, HHC), BF16),
            pltpu.VMEM((CHC, HHC), BF16),
            pltpu.VMEM((CHC, HHC), BF16),
            pltpu.VMEM((CHC, 16), F32),
            pltpu.VMEM((CHC, 16), F32),
            pltpu.VMEM((CHC, 16), F32),
            pltpu.VMEM((CHC, 16), F32),
            pltpu.VMEM((CHC, 16), F32),
            pltpu.VMEM((CHC, 16), F32),
            pltpu.VMEM((CHC, 16), F32),
            pltpu.VMEM((CHC, C), F32),
            pltpu.VMEM((CBLK, CHC), I32),
            pltpu.VMEM((CBLK, CHC), I32),
            pltpu.SemaphoreType.DMA,
            pltpu.SemaphoreType.DMA,
            pltpu.VMEM_SHARED((N, C), F32),
        ],
        **_SC_PARAMS,
    )


def _prep_body(x_ref, wl_ref, wr_ref, pa_ref, pb_ref, qa_ref, qb_ref):
    xv = x_ref[...]
    xl = jnp.dot(xv, wl_ref[...], preferred_element_type=F32)
    xr = jnp.dot(xv, wr_ref[...], preferred_element_type=F32)
    pa_ref[...] = xl[:, :HHC].astype(BF16)
    pb_ref[...] = xl[:, HHC:].astype(BF16)
    qa_ref[...] = xr[:, :HHC].astype(BF16)
    qb_ref[...] = xr[:, HHC:].astype(BF16)


_prep = pl.pallas_call(
    _prep_body,
    grid=(25,),
    in_specs=[
        pl.BlockSpec((N // 25, C), lambda i: (i, 0)),
        pl.BlockSpec((C, HC), lambda i: (0, 0)),
        pl.BlockSpec((C, HC), lambda i: (0, 0)),
    ],
    out_specs=[pl.BlockSpec((N // 25, HHC), lambda i: (i, 0))] * 4,
    out_shape=[jax.ShapeDtypeStruct((N, HHC), BF16)] * 4,
)


def _combine_body(a_ref, b_ref, bias_ref, o_ref):
    s = a_ref[...] + b_ref[...] + bias_ref[...]
    o_ref[...] = jnp.maximum(s, 0.01 * s)


_combine = pl.pallas_call(
    _combine_body,
    grid=(10,),
    in_specs=[
        pl.BlockSpec((N // 10, C), lambda i: (i, 0)),
        pl.BlockSpec((N // 10, C), lambda i: (i, 0)),
        pl.BlockSpec((1, C), lambda i: (0, 0)),
    ],
    out_specs=pl.BlockSpec((N // 10, C), lambda i: (i, 0)),
    out_shape=jax.ShapeDtypeStruct((N, C), F32),
)


def _pool_body(h_ref, batch_ref, wc_ref, bc_ref, o_ref):
    hv = h_ref[...]
    bt = batch_ref[...]
    gids = lax.broadcasted_iota(I32, (N, NGRAPH), 1)
    oh = (bt == gids).astype(F32)
    sums = lax.dot_general(oh, hv, (((0,), (0,)), ((), ())),
                           preferred_element_type=F32)
    counts = jnp.sum(oh, axis=0)
    pooled = sums / jnp.maximum(counts, 1.0)[:, None]
    o_ref[...] = jnp.dot(pooled, wc_ref[...], preferred_element_type=F32) + bc_ref[...]


_pool = pl.pallas_call(
    _pool_body,
    out_shape=jax.ShapeDtypeStruct((NGRAPH, NCLASS), F32),
)


_PI = np.arange(C).reshape(C // 32, 16, 2).transpose(0, 2, 1).reshape(C)


def _gat_layer(h, edges, Wl, Wr, att, b_perm, denz, outz):
    srcg, dstg, srcc, dstc, dst = edges
    pa16, pb16, qa16, qb16 = _prep(h, Wl, Wr)
    attp = att.reshape(HC // 32, 16, 2).transpose(0, 2, 1).reshape(HC)
    lg, pm = _phase_a()(pa16, pb16, qa16, qb16, attp, srcg, dstg)
    ex, d0, d1 = _phase_b()(lg, pm, dst, denz)
    o0, o1 = _phase_c()(pa16, pb16, ex, d0, d1, srcc, dstc, outz)
    return _combine(o0, o1, b_perm.reshape(1, C))


def kernel(x, edge_index, batch, Wl1, Wr1, att1, b1, Wl2, Wr2, att2, b2, Wc, bc):
    src = edge_index[0]
    dst = edge_index[1]
    edges = (src.reshape(E // 16, 16), dst.reshape(E // 16, 16),
             src.reshape(E // CHC, CHC), dst.reshape(E // CHC, CHC), dst)
    denz = jnp.zeros((N, 16), F32)
    outz = jnp.zeros((N, C), F32)
    h = _gat_layer(x, edges, Wl1, Wr1, att1, b1[_PI], denz, outz)
    h = _gat_layer(h, edges, Wl2[_PI, :], Wr2[_PI, :], att2, b2[_PI], denz, outz)
    return _pool(h, batch.reshape(N, 1).astype(I32), Wc[_PI, :],
                 bc.reshape(1, NCLASS))


# speedup vs baseline: 1.1115x; 1.1115x over previous
import jax
import jax.numpy as jnp
import numpy as np
from jax import lax
from jax.experimental import pallas as pl
from jax.experimental.pallas import tpu as pltpu
from jax.experimental.pallas import tpu_sc as plsc

N = 10000
E = 320000
H = 16
C = 128
HC = 2048
HHC = HC // 2
NCLASS = 16
NGRAPH = 64

NW = 32
EPW = E // NW
CH = 16
NCHUNK = EPW // CH
CHB = 80
NCHB = EPW // CHB
CHC = 8
NCHC = EPW // CHC
NBLK = 5
CBLK = NCHC // NBLK
NGRP = E // 16

F32 = jnp.float32
I32 = jnp.int32
BF16 = jnp.bfloat16

_SC_PARAMS = dict(
    mesh=plsc.VectorSubcoreMesh(core_axis_name="c", subcore_axis_name="s"),
    compiler_params=pltpu.CompilerParams(
        use_tc_tiling_on_sc=False, needs_layout_passes=False),
)


def _worker():
    cid = lax.axis_index("c")
    sid = lax.axis_index("s")
    return cid, sid, sid * 2 + cid


def _phase_a_body(xla, xlb, xra, xrb, att_hbm, srcg, dstg, lg_hbm, pm_hbm,
                  pb0, pb1, qb0, qb1, attv, lgbuf, maxbuf, sallv, dallv,
                  sem0, sem1):
    _, _, wid = _worker()
    gbase = wid * NCHUNK
    pltpu.sync_copy(att_hbm, attv)
    pltpu.sync_copy(srcg.at[pl.ds(gbase, NCHUNK)], sallv)
    pltpu.sync_copy(dstg.at[pl.ds(gbase, NCHUNK)], dallv)
    row_iota = lax.iota(I32, 16)
    zero16 = jnp.zeros((16,), F32)
    for h in range(H):
        maxbuf[h, :] = jnp.full((16,), -jnp.inf, F32)

    pbufs, qbufs, sems = (pb0, pb1), (qb0, qb1), (sem0, sem1)
    xlh, xrh = (xla, xlb), (xra, xrb)

    def issue(ci, k, b):
        pltpu.async_copy(xlh[k].at[sallv.at[ci]], pbufs[b], sems[b])
        pltpu.async_copy(xrh[k].at[dallv.at[ci]], qbufs[b], sems[b])

    def wait(b):
        pltpu.make_async_copy(xla.at[sallv.at[0]], pbufs[b], sems[b]).wait()
        pltpu.make_async_copy(xra.at[dallv.at[0]], qbufs[b], sems[b]).wait()

    def compute_half(b, k):
        pbuf, qbuf = pbufs[b], qbufs[b]
        ilv = plsc.PackFormat.INTERLEAVED

        @pl.loop(0, CH, init_carry=tuple(zero16 for _ in range(8)))
        def _edges(e, lgvs):
            oh = jnp.where(row_iota == e, 1.0, 0.0).astype(F32)
            new = []
            for hl in range(8):
                acc = zero16
                for g in range(4):
                    off = hl * 128 + g * 32
                    p0, p1 = plsc.unpack(pbuf[e, pl.ds(off, 32)], format=ilv)
                    q0, q1 = plsc.unpack(qbuf[e, pl.ds(off, 32)], format=ilv)
                    a0, a1 = plsc.unpack(attv[pl.ds(k * HHC + off, 32)], format=ilv)
                    z0 = p0 + q0
                    z1 = p1 + q1
                    acc = (acc + jnp.maximum(z0, 0.2 * z0) * a0
                           + jnp.maximum(z1, 0.2 * z1) * a1)
                new.append(lgvs[hl] + jnp.sum(acc) * oh)
            return tuple(new)
        return _edges

    issue(0, 0, 0)

    @pl.loop(0, NCHUNK)
    def _chunk(i):
        issue(i, 1, 1)
        wait(0)
        lo = compute_half(0, 0)

        @pl.when(i + 1 < NCHUNK)
        def _():
            issue(i + 1, 0, 0)
        wait(1)
        hi = compute_half(1, 1)
        for hl in range(8):
            lgbuf[hl, :] = lo[hl]
            maxbuf[hl, :] = jnp.maximum(maxbuf[hl, :], lo[hl])
            lgbuf[8 + hl, :] = hi[hl]
            maxbuf[8 + hl, :] = jnp.maximum(maxbuf[8 + hl, :], hi[hl])
        pltpu.sync_copy(lgbuf, lg_hbm.at[gbase + i])

    pltpu.sync_copy(maxbuf, pm_hbm.at[wid])


def _phase_a():
    return pl.kernel(
        _phase_a_body,
        out_type=[jax.ShapeDtypeStruct((NGRP, 16, 16), F32),
                  jax.ShapeDtypeStruct((NW, 16, 16), F32)],
        scratch_types=[
            pltpu.VMEM((CH, HHC), BF16),
            pltpu.VMEM((CH, HHC), BF16),
            pltpu.VMEM((CH, HHC), BF16),
            pltpu.VMEM((CH, HHC), BF16),
            pltpu.VMEM((HC,), BF16),
            pltpu.VMEM((16, 16), F32),
            pltpu.VMEM((16, 16), F32),
            pltpu.VMEM((NCHUNK, CH), I32),
            pltpu.VMEM((NCHUNK, CH), I32),
            pltpu.SemaphoreType.DMA,
            pltpu.SemaphoreType.DMA,
        ],
        **_SC_PARAMS,
    )


def _phase_b_body(lg_hbm, pm_hbm, dst_hbm, denz_hbm, ex_hbm, den0_hbm, den1_hbm,
                  pmall, gsplat, lgc, tmph, exbuf, didx, densh):
    cid, sid, wid = _worker()
    ebase = wid * EPW
    row_iota = lax.iota(I32, 16)

    @pl.when(sid == 0)
    def _():
        pltpu.sync_copy(denz_hbm, densh)
    plsc.subcore_barrier()

    pltpu.sync_copy(pm_hbm, pmall)
    for h in range(H):
        gv = pmall[0, h, :]
        for w in range(1, NW):
            gv = jnp.maximum(gv, pmall[w, h, :])
        gsplat[h, :] = jnp.full((16,), jnp.max(gv), F32)

    @pl.loop(0, NCHB)
    def _chunk(i):
        base = ebase + i * CHB
        g0 = ebase // 16 + i * (CHB // 16)
        pltpu.sync_copy(lg_hbm.at[pl.ds(g0, CHB // 16)], lgc)
        pltpu.sync_copy(dst_hbm.at[pl.ds(base, CHB)], didx)
        for g in range(CHB // 16):
            for h in range(H):
                tmph[h, :] = jnp.exp(lgc[g, h, :] - gsplat[h, :])
            for e in range(16):
                esplat = jnp.full((16,), e, I32)
                exbuf[g * 16 + e, :] = plsc.load_gather(tmph, [row_iota, esplat])
        pltpu.sync_copy(exbuf, ex_hbm.at[pl.ds(base, CHB)])
        pltpu.sync_copy(exbuf, densh.at[didx], add=True)

    plsc.subcore_barrier()

    @pl.when(jnp.logical_and(sid == 0, cid == 0))
    def _():
        pltpu.sync_copy(densh, den0_hbm)

    @pl.when(jnp.logical_and(sid == 0, cid == 1))
    def _():
        pltpu.sync_copy(densh, den1_hbm)


def _phase_b():
    return pl.kernel(
        _phase_b_body,
        out_type=[jax.ShapeDtypeStruct((E, 16), F32),
                  jax.ShapeDtypeStruct((N, 16), F32),
                  jax.ShapeDtypeStruct((N, 16), F32)],
        scratch_types=[
            pltpu.VMEM((NW, 16, 16), F32),
            pltpu.VMEM((16, 16), F32),
            pltpu.VMEM((CHB // 16, 16, 16), F32),
            pltpu.VMEM((16, 16), F32),
            pltpu.VMEM((CHB, 16), F32),
            pltpu.VMEM((CHB,), I32),
            pltpu.VMEM_SHARED((N, 16), F32),
        ],
        **_SC_PARAMS,
    )


def _phase_c_body(xla, xlb, ex_hbm, den0_hbm, den1_hbm, srcc, dstc,
                  outz_hbm, out0_hbm, out1_hbm,
                  xba0, xba1, xbb0, xbb1, exb0, exb1, d0b0, d0b1, d1b0, d1b1,
                  abuf, wbuf, sallv, dallv, sem0, sem1, outsh):
    cid, sid, wid = _worker()
    ebase = wid * EPW

    @pl.when(sid == 0)
    def _():
        pltpu.sync_copy(outz_hbm, outsh)
    plsc.subcore_barrier()

    xbas, xbbs = (xba0, xba1), (xbb0, xbb1)
    exbs, d0bs, d1bs = (exb0, exb1), (d0b0, d0b1), (d1b0, d1b1)
    sems = (sem0, sem1)

    def issue(blk, ci, b):
        base = ebase + (blk * CBLK + ci) * CHC
        pltpu.async_copy(xla.at[sallv.at[ci]], xbas[b], sems[b])
        pltpu.async_copy(xlb.at[sallv.at[ci]], xbbs[b], sems[b])
        pltpu.async_copy(ex_hbm.at[pl.ds(base, CHC)], exbs[b], sems[b])
        pltpu.async_copy(den0_hbm.at[dallv.at[ci]], d0bs[b], sems[b])
        pltpu.async_copy(den1_hbm.at[dallv.at[ci]], d1bs[b], sems[b])

    def wait(b):
        pltpu.make_async_copy(xla.at[sallv.at[0]], xbas[b], sems[b]).wait()
        pltpu.make_async_copy(xlb.at[sallv.at[0]], xbbs[b], sems[b]).wait()
        pltpu.make_async_copy(ex_hbm.at[pl.ds(ebase, CHC)], exbs[b], sems[b]).wait()
        pltpu.make_async_copy(den0_hbm.at[dallv.at[0]], d0bs[b], sems[b]).wait()
        pltpu.make_async_copy(den1_hbm.at[dallv.at[0]], d1bs[b], sems[b]).wait()

    def compute(ci, b):
        xba, xbb, exb, d0b, d1b = xbas[b], xbbs[b], exbs[b], d0bs[b], d1bs[b]
        ilv = plsc.PackFormat.INTERLEAVED
        for e in range(CHC):
            d = (d0b[e, :] + d1b[e, :]) * float(H)
            abuf[e, :] = exb[e, :] / d
        for e in range(CHC):
            zeros8 = tuple(jnp.zeros((16,), F32) for _ in range(8))
            esplat = jnp.full((16,), e, I32)

            @pl.loop(0, 8, init_carry=zeros8)
            def _lo(hh, accs):
                alv = plsc.load_gather(abuf, [esplat, jnp.full((16,), hh, I32)])
                out = list(accs)
                for g in range(4):
                    x0, x1 = plsc.unpack(
                        xba[e, pl.ds(hh * 128 + g * 32, 32)], format=ilv)
                    out[2 * g] = out[2 * g] + alv * x0
                    out[2 * g + 1] = out[2 * g + 1] + alv * x1
                return tuple(out)

            @pl.loop(8, 16, init_carry=_lo)
            def _hi(hh, accs):
                alv = plsc.load_gather(abuf, [esplat, jnp.full((16,), hh, I32)])
                out = list(accs)
                for g in range(4):
                    x0, x1 = plsc.unpack(
                        xbb[e, pl.ds((hh - 8) * 128 + g * 32, 32)], format=ilv)
                    out[2 * g] = out[2 * g] + alv * x0
                    out[2 * g + 1] = out[2 * g + 1] + alv * x1
                return tuple(out)

            for g in range(4):
                wbuf[e, pl.ds(g * 32, 16)] = _hi[2 * g]
                wbuf[e, pl.ds(g * 32 + 16, 16)] = _hi[2 * g + 1]
        pltpu.sync_copy(wbuf, outsh.at[dallv.at[ci]], add=True)

    for blk in range(NBLK):
        cb0 = wid * NCHC + blk * CBLK
        pltpu.sync_copy(srcc.at[pl.ds(cb0, CBLK)], sallv)
        pltpu.sync_copy(dstc.at[pl.ds(cb0, CBLK)], dallv)
        issue(blk, 0, 0)

        @pl.loop(0, CBLK // 2)
        def _pair(p):
            i0 = p * 2
            issue(blk, i0 + 1, 1)
            wait(0)
            compute(i0, 0)

            @pl.when(i0 + 2 < CBLK)
            def _():
                issue(blk, i0 + 2, 0)
            wait(1)
            compute(i0 + 1, 1)

    plsc.subcore_barrier()

    @pl.when(jnp.logical_and(sid == 0, cid == 0))
    def _():
        pltpu.sync_copy(outsh, out0_hbm)

    @pl.when(jnp.logical_and(sid == 0, cid == 1))
    def _():
        pltpu.sync_copy(outsh, out1_hbm)


def _phase_c():
    return pl.kernel(
        _phase_c_body,
        out_type=[jax.ShapeDtypeStruct((N, C), F32),
                  jax.ShapeDtypeStruct((N, C), F32)],
        scratch_types=[
            pltpu.VMEM((CHC, HHC), BF16),
            pltpu.VMEM((CHC, HHC), BF16),
            pltpu.VMEM((CHC, HHC), BF16),
            pltpu.VMEM((CHC, HHC), BF16),
            pltpu.VMEM((CHC, 16), F32),
            pltpu.VMEM((CHC, 16), F32),
            pltpu.VMEM((CHC, 16), F32),
            pltpu.VMEM((CHC, 16), F32),
            pltpu.VMEM((CHC, 16), F32),
            pltpu.VMEM((CHC, 16), F32),
            pltpu.VMEM((CHC, 16), F32),
            pltpu.VMEM((CHC, C), F32),
            pltpu.VMEM((CBLK, CHC), I32),
            pltpu.VMEM((CBLK, CHC), I32),
            pltpu.SemaphoreType.DMA,
            pltpu.SemaphoreType.DMA,
            pltpu.VMEM_SHARED((N, C), F32),
        ],
        **_SC_PARAMS,
    )


def _prep_body(x_ref, wl_ref, wr_ref, pa_ref, pb_ref, qa_ref, qb_ref):
    xv = x_ref[...]
    xl = jnp.dot(xv, wl_ref[...], preferred_element_type=F32)
    xr = jnp.dot(xv, wr_ref[...], preferred_element_type=F32)
    pa_ref[...] = xl[:, :HHC].astype(BF16)
    pb_ref[...] = xl[:, HHC:].astype(BF16)
    qa_ref[...] = xr[:, :HHC].astype(BF16)
    qb_ref[...] = xr[:, HHC:].astype(BF16)


_prep = pl.pallas_call(
    _prep_body,
    grid=(25,),
    in_specs=[
        pl.BlockSpec((N // 25, C), lambda i: (i, 0)),
        pl.BlockSpec((C, HC), lambda i: (0, 0)),
        pl.BlockSpec((C, HC), lambda i: (0, 0)),
    ],
    out_specs=[pl.BlockSpec((N // 25, HHC), lambda i: (i, 0))] * 4,
    out_shape=[jax.ShapeDtypeStruct((N, HHC), BF16)] * 4,
)


def _combine_body(a_ref, b_ref, bias_ref, o_ref):
    s = a_ref[...] + b_ref[...] + bias_ref[...]
    o_ref[...] = jnp.maximum(s, 0.01 * s)


_combine = pl.pallas_call(
    _combine_body,
    grid=(10,),
    in_specs=[
        pl.BlockSpec((N // 10, C), lambda i: (i, 0)),
        pl.BlockSpec((N // 10, C), lambda i: (i, 0)),
        pl.BlockSpec((1, C), lambda i: (0, 0)),
    ],
    out_specs=pl.BlockSpec((N // 10, C), lambda i: (i, 0)),
    out_shape=jax.ShapeDtypeStruct((N, C), F32),
)


def _pool_body(h_ref, batch_ref, wc_ref, bc_ref, o_ref):
    hv = h_ref[...]
    bt = batch_ref[...]
    gids = lax.broadcasted_iota(I32, (N, NGRAPH), 1)
    oh = (bt == gids).astype(F32)
    sums = lax.dot_general(oh, hv, (((0,), (0,)), ((), ())),
                           preferred_element_type=F32)
    counts = jnp.sum(oh, axis=0)
    pooled = sums / jnp.maximum(counts, 1.0)[:, None]
    o_ref[...] = jnp.dot(pooled, wc_ref[...], preferred_element_type=F32) + bc_ref[...]


_pool = pl.pallas_call(
    _pool_body,
    out_shape=jax.ShapeDtypeStruct((NGRAPH, NCLASS), F32),
)


_PI = np.arange(C).reshape(C // 32, 16, 2).transpose(0, 2, 1).reshape(C)


def _gat_layer(h, edges, Wl, Wr, att, b_perm, denz, outz):
    srcg, dstg, srcc, dstc, dst = edges
    pa16, pb16, qa16, qb16 = _prep(h, Wl, Wr)
    att16 = att.reshape(HC).astype(BF16)
    lg, pm = _phase_a()(pa16, pb16, qa16, qb16, att16, srcg, dstg)
    ex, d0, d1 = _phase_b()(lg, pm, dst, denz)
    o0, o1 = _phase_c()(pa16, pb16, ex, d0, d1, srcc, dstc, outz)
    return _combine(o0, o1, b_perm.reshape(1, C))


def kernel(x, edge_index, batch, Wl1, Wr1, att1, b1, Wl2, Wr2, att2, b2, Wc, bc):
    src = edge_index[0]
    dst = edge_index[1]
    edges = (src.reshape(E // 16, 16), dst.reshape(E // 16, 16),
             src.reshape(E // CHC, CHC), dst.reshape(E // CHC, CHC), dst)
    denz = jnp.zeros((N, 16), F32)
    outz = jnp.zeros((N, C), F32)
    h = _gat_layer(x, edges, Wl1, Wr1, att1, b1[_PI], denz, outz)
    h = _gat_layer(h, edges, Wl2[_PI, :], Wr2[_PI, :], att2, b2[_PI], denz, outz)
    return _pool(h, batch.reshape(N, 1).astype(I32), Wc[_PI, :],
                 bc.reshape(1, NCLASS))


# speedup vs baseline: 1.1485x; 1.0333x over previous
import jax
import jax.numpy as jnp
import numpy as np
from jax import lax
from jax.experimental import pallas as pl
from jax.experimental.pallas import tpu as pltpu
from jax.experimental.pallas import tpu_sc as plsc

N = 10000
E = 320000
H = 16
C = 128
HC = 2048
HHC = HC // 2
NCLASS = 16
NGRAPH = 64

NW = 32
EPW = E // NW
CH = 16
NCHUNK = EPW // CH
CHB = 80
NCHB = EPW // CHB
CHC = 10
NCHC = EPW // CHC
NBLK = 5
CBLK = NCHC // NBLK
NGRP = E // 16

F32 = jnp.float32
I32 = jnp.int32
BF16 = jnp.bfloat16

_SC_PARAMS = dict(
    mesh=plsc.VectorSubcoreMesh(core_axis_name="c", subcore_axis_name="s"),
    compiler_params=pltpu.CompilerParams(
        use_tc_tiling_on_sc=False, needs_layout_passes=False),
)


def _worker():
    cid = lax.axis_index("c")
    sid = lax.axis_index("s")
    return cid, sid, sid * 2 + cid


def _phase_a_body(xla, xlb, xra, xrb, att_hbm, srcg, dstg, lg_hbm, pm_hbm,
                  pb0, pb1, qb0, qb1, attv, lgbuf, maxbuf, sallv, dallv,
                  sem0, sem1):
    _, _, wid = _worker()
    gbase = wid * NCHUNK
    pltpu.sync_copy(att_hbm, attv)
    pltpu.sync_copy(srcg.at[pl.ds(gbase, NCHUNK)], sallv)
    pltpu.sync_copy(dstg.at[pl.ds(gbase, NCHUNK)], dallv)
    row_iota = lax.iota(I32, 16)
    zero16 = jnp.zeros((16,), F32)
    for h in range(H):
        maxbuf[h, :] = jnp.full((16,), -jnp.inf, F32)

    pbufs, qbufs, sems = (pb0, pb1), (qb0, qb1), (sem0, sem1)
    xlh, xrh = (xla, xlb), (xra, xrb)

    def issue(ci, k, b):
        pltpu.async_copy(xlh[k].at[sallv.at[ci]], pbufs[b], sems[b])
        pltpu.async_copy(xrh[k].at[dallv.at[ci]], qbufs[b], sems[b])

    def wait(b):
        pltpu.make_async_copy(xla.at[sallv.at[0]], pbufs[b], sems[b]).wait()
        pltpu.make_async_copy(xra.at[dallv.at[0]], qbufs[b], sems[b]).wait()

    def compute_half(b, k):
        pbuf, qbuf = pbufs[b], qbufs[b]
        ilv = plsc.PackFormat.INTERLEAVED

        @pl.loop(0, CH, init_carry=tuple(zero16 for _ in range(8)))
        def _edges(e, lgvs):
            oh = jnp.where(row_iota == e, 1.0, 0.0).astype(F32)
            new = []
            for hl in range(8):
                acc = zero16
                for g in range(4):
                    off = hl * 128 + g * 32
                    p0, p1 = plsc.unpack(pbuf[e, pl.ds(off, 32)], format=ilv)
                    q0, q1 = plsc.unpack(qbuf[e, pl.ds(off, 32)], format=ilv)
                    a0, a1 = plsc.unpack(attv[pl.ds(k * HHC + off, 32)], format=ilv)
                    z0 = p0 + q0
                    z1 = p1 + q1
                    acc = (acc + jnp.maximum(z0, 0.2 * z0) * a0
                           + jnp.maximum(z1, 0.2 * z1) * a1)
                new.append(lgvs[hl] + jnp.sum(acc) * oh)
            return tuple(new)
        return _edges

    issue(0, 0, 0)

    @pl.loop(0, NCHUNK)
    def _chunk(i):
        issue(i, 1, 1)
        wait(0)
        lo = compute_half(0, 0)

        @pl.when(i + 1 < NCHUNK)
        def _():
            issue(i + 1, 0, 0)
        wait(1)
        hi = compute_half(1, 1)
        for hl in range(8):
            lgbuf[hl, :] = lo[hl]
            maxbuf[hl, :] = jnp.maximum(maxbuf[hl, :], lo[hl])
            lgbuf[8 + hl, :] = hi[hl]
            maxbuf[8 + hl, :] = jnp.maximum(maxbuf[8 + hl, :], hi[hl])
        pltpu.sync_copy(lgbuf, lg_hbm.at[gbase + i])

    pltpu.sync_copy(maxbuf, pm_hbm.at[wid])


def _phase_a():
    return pl.kernel(
        _phase_a_body,
        out_type=[jax.ShapeDtypeStruct((NGRP, 16, 16), F32),
                  jax.ShapeDtypeStruct((NW, 16, 16), F32)],
        scratch_types=[
            pltpu.VMEM((CH, HHC), BF16),
            pltpu.VMEM((CH, HHC), BF16),
            pltpu.VMEM((CH, HHC), BF16),
            pltpu.VMEM((CH, HHC), BF16),
            pltpu.VMEM((HC,), BF16),
            pltpu.VMEM((16, 16), F32),
            pltpu.VMEM((16, 16), F32),
            pltpu.VMEM((NCHUNK, CH), I32),
            pltpu.VMEM((NCHUNK, CH), I32),
            pltpu.SemaphoreType.DMA,
            pltpu.SemaphoreType.DMA,
        ],
        **_SC_PARAMS,
    )


def _phase_b_body(lg_hbm, pm_hbm, dst_hbm, denz_hbm, ex_hbm, den0_hbm, den1_hbm,
                  pmall, gsplat, lgc, tmph, exbuf, didx, densh):
    cid, sid, wid = _worker()
    ebase = wid * EPW
    row_iota = lax.iota(I32, 16)

    @pl.when(sid == 0)
    def _():
        pltpu.sync_copy(denz_hbm, densh)
    plsc.subcore_barrier()

    pltpu.sync_copy(pm_hbm, pmall)
    for h in range(H):
        gv = pmall[0, h, :]
        for w in range(1, NW):
            gv = jnp.maximum(gv, pmall[w, h, :])
        gsplat[h, :] = jnp.full((16,), jnp.max(gv), F32)

    @pl.loop(0, NCHB)
    def _chunk(i):
        base = ebase + i * CHB
        g0 = ebase // 16 + i * (CHB // 16)
        pltpu.sync_copy(lg_hbm.at[pl.ds(g0, CHB // 16)], lgc)
        pltpu.sync_copy(dst_hbm.at[pl.ds(base, CHB)], didx)
        for g in range(CHB // 16):
            for h in range(H):
                tmph[h, :] = jnp.exp(lgc[g, h, :] - gsplat[h, :])
            for e in range(16):
                esplat = jnp.full((16,), e, I32)
                exbuf[g * 16 + e, :] = plsc.load_gather(tmph, [row_iota, esplat])
        pltpu.sync_copy(exbuf, ex_hbm.at[pl.ds(base, CHB)])
        pltpu.sync_copy(exbuf, densh.at[didx], add=True)

    plsc.subcore_barrier()

    @pl.when(jnp.logical_and(sid == 0, cid == 0))
    def _():
        pltpu.sync_copy(densh, den0_hbm)

    @pl.when(jnp.logical_and(sid == 0, cid == 1))
    def _():
        pltpu.sync_copy(densh, den1_hbm)


def _phase_b():
    return pl.kernel(
        _phase_b_body,
        out_type=[jax.ShapeDtypeStruct((E, 16), F32),
                  jax.ShapeDtypeStruct((N, 16), F32),
                  jax.ShapeDtypeStruct((N, 16), F32)],
        scratch_types=[
            pltpu.VMEM((NW, 16, 16), F32),
            pltpu.VMEM((16, 16), F32),
            pltpu.VMEM((CHB // 16, 16, 16), F32),
            pltpu.VMEM((16, 16), F32),
            pltpu.VMEM((CHB, 16), F32),
            pltpu.VMEM((CHB,), I32),
            pltpu.VMEM_SHARED((N, 16), F32),
        ],
        **_SC_PARAMS,
    )


def _phase_c_body(xla, xlb, ex_hbm, den0_hbm, den1_hbm, srcc, dstc,
                  outz_hbm, out0_hbm, out1_hbm,
                  xba0, xba1, xbb0, xbb1, exb0, exb1, d0b0, d0b1, d1b0, d1b1,
                  abuf, wbuf0, wbuf1, sallv, dallv, sem0, sem1, semw0, semw1,
                  outsh):
    cid, sid, wid = _worker()
    ebase = wid * EPW

    @pl.when(sid == 0)
    def _():
        pltpu.sync_copy(outz_hbm, outsh)
    plsc.subcore_barrier()

    xbas, xbbs = (xba0, xba1), (xbb0, xbb1)
    exbs, d0bs, d1bs = (exb0, exb1), (d0b0, d0b1), (d1b0, d1b1)
    sems = (sem0, sem1)
    wbufs, semw = (wbuf0, wbuf1), (semw0, semw1)

    def issue(blk, ci, b):
        base = ebase + (blk * CBLK + ci) * CHC
        pltpu.async_copy(xla.at[sallv.at[ci]], xbas[b], sems[b])
        pltpu.async_copy(xlb.at[sallv.at[ci]], xbbs[b], sems[b])
        pltpu.async_copy(ex_hbm.at[pl.ds(base, CHC)], exbs[b], sems[b])
        pltpu.async_copy(den0_hbm.at[dallv.at[ci]], d0bs[b], sems[b])
        pltpu.async_copy(den1_hbm.at[dallv.at[ci]], d1bs[b], sems[b])

    def wait(b):
        pltpu.make_async_copy(xla.at[sallv.at[0]], xbas[b], sems[b]).wait()
        pltpu.make_async_copy(xlb.at[sallv.at[0]], xbbs[b], sems[b]).wait()
        pltpu.make_async_copy(ex_hbm.at[pl.ds(ebase, CHC)], exbs[b], sems[b]).wait()
        pltpu.make_async_copy(den0_hbm.at[dallv.at[0]], d0bs[b], sems[b]).wait()
        pltpu.make_async_copy(den1_hbm.at[dallv.at[0]], d1bs[b], sems[b]).wait()

    def compute(blk, ci, b):
        xba, xbb, exb, d0b, d1b = xbas[b], xbbs[b], exbs[b], d0bs[b], d1bs[b]
        wbuf = wbufs[b]
        ilv = plsc.PackFormat.INTERLEAVED

        @pl.loop(0, CHC)
        def _alpha(e):
            d = (d0b[e, :] + d1b[e, :]) * float(H)
            abuf[e, :] = exb[e, :] / d

        @pl.loop(0, CHC)
        def _edge(e):
            zeros8 = tuple(jnp.zeros((16,), F32) for _ in range(8))
            esplat = jnp.full((16,), e, I32)

            @pl.loop(0, 8, init_carry=zeros8)
            def _lo(hh, accs):
                alv = plsc.load_gather(abuf, [esplat, jnp.full((16,), hh, I32)])
                out = list(accs)
                for g in range(4):
                    x0, x1 = plsc.unpack(
                        xba[e, pl.ds(hh * 128 + g * 32, 32)], format=ilv)
                    out[2 * g] = out[2 * g] + alv * x0
                    out[2 * g + 1] = out[2 * g + 1] + alv * x1
                return tuple(out)

            @pl.loop(8, 16, init_carry=_lo)
            def _hi(hh, accs):
                alv = plsc.load_gather(abuf, [esplat, jnp.full((16,), hh, I32)])
                out = list(accs)
                for g in range(4):
                    x0, x1 = plsc.unpack(
                        xbb[e, pl.ds((hh - 8) * 128 + g * 32, 32)], format=ilv)
                    out[2 * g] = out[2 * g] + alv * x0
                    out[2 * g + 1] = out[2 * g + 1] + alv * x1
                return tuple(out)

            for g in range(4):
                wbuf[e, pl.ds(g * 32, 16)] = _hi[2 * g]
                wbuf[e, pl.ds(g * 32 + 16, 16)] = _hi[2 * g + 1]
        pltpu.sync_copy(wbuf, outsh.at[dallv.at[ci]], add=True)

    for blk in range(NBLK):
        cb0 = wid * NCHC + blk * CBLK
        pltpu.sync_copy(srcc.at[pl.ds(cb0, CBLK)], sallv)
        pltpu.sync_copy(dstc.at[pl.ds(cb0, CBLK)], dallv)
        issue(blk, 0, 0)

        @pl.loop(0, CBLK // 2)
        def _pair(p):
            i0 = p * 2
            issue(blk, i0 + 1, 1)
            wait(0)
            compute(blk, i0, 0)

            @pl.when(i0 + 2 < CBLK)
            def _():
                issue(blk, i0 + 2, 0)
            wait(1)
            compute(blk, i0 + 1, 1)

    plsc.subcore_barrier()

    @pl.when(jnp.logical_and(sid == 0, cid == 0))
    def _():
        pltpu.sync_copy(outsh, out0_hbm)

    @pl.when(jnp.logical_and(sid == 0, cid == 1))
    def _():
        pltpu.sync_copy(outsh, out1_hbm)


def _phase_c():
    return pl.kernel(
        _phase_c_body,
        out_type=[jax.ShapeDtypeStruct((N, C), F32),
                  jax.ShapeDtypeStruct((N, C), F32)],
        scratch_types=[
            pltpu.VMEM((CHC, HHC), BF16),
            pltpu.VMEM((CHC, HHC), BF16),
            pltpu.VMEM((CHC, HHC), BF16),
            pltpu.VMEM((CHC, HHC), BF16),
            pltpu.VMEM((CHC, 16), F32),
            pltpu.VMEM((CHC, 16), F32),
            pltpu.VMEM((CHC, 16), F32),
            pltpu.VMEM((CHC, 16), F32),
            pltpu.VMEM((CHC, 16), F32),
            pltpu.VMEM((CHC, 16), F32),
            pltpu.VMEM((CHC, 16), F32),
            pltpu.VMEM((CHC, C), F32),
            pltpu.VMEM((CHC, C), F32),
            pltpu.VMEM((CBLK, CHC), I32),
            pltpu.VMEM((CBLK, CHC), I32),
            pltpu.SemaphoreType.DMA,
            pltpu.SemaphoreType.DMA,
            pltpu.SemaphoreType.DMA,
            pltpu.SemaphoreType.DMA,
            pltpu.VMEM_SHARED((N, C), F32),
        ],
        **_SC_PARAMS,
    )


def _prep_body(x_ref, wl_ref, wr_ref, pa_ref, pb_ref, qa_ref, qb_ref):
    xv = x_ref[...]
    xl = jnp.dot(xv, wl_ref[...], preferred_element_type=F32)
    xr = jnp.dot(xv, wr_ref[...], preferred_element_type=F32)
    pa_ref[...] = xl[:, :HHC].astype(BF16)
    pb_ref[...] = xl[:, HHC:].astype(BF16)
    qa_ref[...] = xr[:, :HHC].astype(BF16)
    qb_ref[...] = xr[:, HHC:].astype(BF16)


_prep = pl.pallas_call(
    _prep_body,
    grid=(25,),
    in_specs=[
        pl.BlockSpec((N // 25, C), lambda i: (i, 0)),
        pl.BlockSpec((C, HC), lambda i: (0, 0)),
        pl.BlockSpec((C, HC), lambda i: (0, 0)),
    ],
    out_specs=[pl.BlockSpec((N // 25, HHC), lambda i: (i, 0))] * 4,
    out_shape=[jax.ShapeDtypeStruct((N, HHC), BF16)] * 4,
)


def _combine_body(a_ref, b_ref, bias_ref, o_ref):
    s = a_ref[...] + b_ref[...] + bias_ref[...]
    o_ref[...] = jnp.maximum(s, 0.01 * s)


_combine = pl.pallas_call(
    _combine_body,
    grid=(10,),
    in_specs=[
        pl.BlockSpec((N // 10, C), lambda i: (i, 0)),
        pl.BlockSpec((N // 10, C), lambda i: (i, 0)),
        pl.BlockSpec((1, C), lambda i: (0, 0)),
    ],
    out_specs=pl.BlockSpec((N // 10, C), lambda i: (i, 0)),
    out_shape=jax.ShapeDtypeStruct((N, C), F32),
)


def _pool_body(h_ref, batch_ref, wc_ref, bc_ref, o_ref):
    hv = h_ref[...]
    bt = batch_ref[...]
    gids = lax.broadcasted_iota(I32, (N, NGRAPH), 1)
    oh = (bt == gids).astype(F32)
    sums = lax.dot_general(oh, hv, (((0,), (0,)), ((), ())),
                           preferred_element_type=F32)
    counts = jnp.sum(oh, axis=0)
    pooled = sums / jnp.maximum(counts, 1.0)[:, None]
    o_ref[...] = jnp.dot(pooled, wc_ref[...], preferred_element_type=F32) + bc_ref[...]


_pool = pl.pallas_call(
    _pool_body,
    out_shape=jax.ShapeDtypeStruct((NGRAPH, NCLASS), F32),
)


_PI = np.arange(C).reshape(C // 32, 16, 2).transpose(0, 2, 1).reshape(C)


def _gat_layer(h, edges, Wl, Wr, att, b_perm, denz, outz):
    srcg, dstg, srcc, dstc, dst = edges
    pa16, pb16, qa16, qb16 = _prep(h, Wl, Wr)
    att16 = att.reshape(HC).astype(BF16)
    lg, pm = _phase_a()(pa16, pb16, qa16, qb16, att16, srcg, dstg)
    ex, d0, d1 = _phase_b()(lg, pm, dst, denz)
    o0, o1 = _phase_c()(pa16, pb16, ex, d0, d1, srcc, dstc, outz)
    return _combine(o0, o1, b_perm.reshape(1, C))


def kernel(x, edge_index, batch, Wl1, Wr1, att1, b1, Wl2, Wr2, att2, b2, Wc, bc):
    src = edge_index[0]
    dst = edge_index[1]
    edges = (src.reshape(E // 16, 16), dst.reshape(E // 16, 16),
             src.reshape(E // CHC, CHC), dst.reshape(E // CHC, CHC), dst)
    denz = jnp.zeros((N, 16), F32)
    outz = jnp.zeros((N, C), F32)
    h = _gat_layer(x, edges, Wl1, Wr1, att1, b1[_PI], denz, outz)
    h = _gat_layer(h, edges, Wl2[_PI, :], Wr2[_PI, :], att2, b2[_PI], denz, outz)
    return _pool(h, batch.reshape(N, 1).astype(I32), Wc[_PI, :],
                 bc.reshape(1, NCLASS))


# speedup vs baseline: 1.1736x; 1.0219x over previous
import jax
import jax.numpy as jnp
import numpy as np
from jax import lax
from jax.experimental import pallas as pl
from jax.experimental.pallas import tpu as pltpu
from jax.experimental.pallas import tpu_sc as plsc

N = 10000
E = 320000
H = 16
C = 128
HC = 2048
HHC = HC // 2
NCLASS = 16
NGRAPH = 64

NW = 32
EPW = E // NW
CH = 16
NCHUNK = EPW // CH
CHB = 80
NCHB = EPW // CHB
CHC = 10
NCHC = EPW // CHC
NBLK = 5
CBLK = NCHC // NBLK
NGRP = E // 16

F32 = jnp.float32
I32 = jnp.int32
BF16 = jnp.bfloat16

_SC_PARAMS = dict(
    mesh=plsc.VectorSubcoreMesh(core_axis_name="c", subcore_axis_name="s"),
    compiler_params=pltpu.CompilerParams(
        use_tc_tiling_on_sc=False, needs_layout_passes=False),
)


def _worker():
    cid = lax.axis_index("c")
    sid = lax.axis_index("s")
    return cid, sid, sid * 2 + cid


def _phase_a_body(xla, xlb, xra, xrb, att_hbm, srcg, dstg, lg_hbm, pm_hbm,
                  pb0, pb1, qb0, qb1, attv, lgbuf, maxbuf, sallv, dallv,
                  sem0, sem1):
    _, _, wid = _worker()
    gbase = wid * NCHUNK
    pltpu.sync_copy(att_hbm, attv)
    pltpu.sync_copy(srcg.at[pl.ds(gbase, NCHUNK)], sallv)
    pltpu.sync_copy(dstg.at[pl.ds(gbase, NCHUNK)], dallv)
    row_iota = lax.iota(I32, 16)
    zero16 = jnp.zeros((16,), F32)
    for h in range(H):
        maxbuf[h, :] = jnp.full((16,), -jnp.inf, F32)

    pbufs, qbufs, sems = (pb0, pb1), (qb0, qb1), (sem0, sem1)
    xlh, xrh = (xla, xlb), (xra, xrb)

    def issue(ci, k, b):
        pltpu.async_copy(xlh[k].at[sallv.at[ci]], pbufs[b], sems[b])
        pltpu.async_copy(xrh[k].at[dallv.at[ci]], qbufs[b], sems[b])

    def wait(b):
        pltpu.make_async_copy(xla.at[sallv.at[0]], pbufs[b], sems[b]).wait()
        pltpu.make_async_copy(xra.at[dallv.at[0]], qbufs[b], sems[b]).wait()

    def compute_half(b, k):
        pbuf, qbuf = pbufs[b], qbufs[b]
        ilv = plsc.PackFormat.INTERLEAVED

        @pl.loop(0, CH, init_carry=tuple(zero16 for _ in range(8)))
        def _edges(e, lgvs):
            oh = jnp.where(row_iota == e, 1.0, 0.0).astype(F32)
            new = []
            for hl in range(8):
                acc = zero16
                for g in range(4):
                    off = hl * 128 + g * 32
                    p0, p1 = plsc.unpack(pbuf[e, pl.ds(off, 32)], format=ilv)
                    q0, q1 = plsc.unpack(qbuf[e, pl.ds(off, 32)], format=ilv)
                    a0, a1 = plsc.unpack(attv[pl.ds(k * HHC + off, 32)], format=ilv)
                    z0 = p0 + q0
                    z1 = p1 + q1
                    acc = (acc + jnp.maximum(z0, 0.2 * z0) * a0
                           + jnp.maximum(z1, 0.2 * z1) * a1)
                new.append(lgvs[hl] + jnp.sum(acc) * oh)
            return tuple(new)
        return _edges

    issue(0, 0, 0)

    @pl.loop(0, NCHUNK)
    def _chunk(i):
        issue(i, 1, 1)
        wait(0)
        lo = compute_half(0, 0)

        @pl.when(i + 1 < NCHUNK)
        def _():
            issue(i + 1, 0, 0)
        wait(1)
        hi = compute_half(1, 1)
        for hl in range(8):
            lgbuf[hl, :] = lo[hl]
            maxbuf[hl, :] = jnp.maximum(maxbuf[hl, :], lo[hl])
            lgbuf[8 + hl, :] = hi[hl]
            maxbuf[8 + hl, :] = jnp.maximum(maxbuf[8 + hl, :], hi[hl])
        pltpu.sync_copy(lgbuf, lg_hbm.at[gbase + i])

    pltpu.sync_copy(maxbuf, pm_hbm.at[wid])


def _phase_a():
    return pl.kernel(
        _phase_a_body,
        out_type=[jax.ShapeDtypeStruct((NGRP, 16, 16), F32),
                  jax.ShapeDtypeStruct((NW, 16, 16), F32)],
        scratch_types=[
            pltpu.VMEM((CH, HHC), BF16),
            pltpu.VMEM((CH, HHC), BF16),
            pltpu.VMEM((CH, HHC), BF16),
            pltpu.VMEM((CH, HHC), BF16),
            pltpu.VMEM((HC,), BF16),
            pltpu.VMEM((16, 16), F32),
            pltpu.VMEM((16, 16), F32),
            pltpu.VMEM((NCHUNK, CH), I32),
            pltpu.VMEM((NCHUNK, CH), I32),
            pltpu.SemaphoreType.DMA,
            pltpu.SemaphoreType.DMA,
        ],
        **_SC_PARAMS,
    )


def _phase_b_body(lg_hbm, pm_hbm, dst_hbm, denz_hbm, ex_hbm, den0_hbm, den1_hbm,
                  pmall, gsplat, lgc, tmph, exbuf, didx, densh):
    cid, sid, wid = _worker()
    ebase = wid * EPW
    row_iota = lax.iota(I32, 16)

    @pl.when(sid == 0)
    def _():
        pltpu.sync_copy(denz_hbm, densh)
    plsc.subcore_barrier()

    pltpu.sync_copy(pm_hbm, pmall)
    for h in range(H):
        gv = pmall[0, h, :]
        for w in range(1, NW):
            gv = jnp.maximum(gv, pmall[w, h, :])
        gsplat[h, :] = jnp.full((16,), jnp.max(gv), F32)

    @pl.loop(0, NCHB)
    def _chunk(i):
        base = ebase + i * CHB
        g0 = ebase // 16 + i * (CHB // 16)
        pltpu.sync_copy(lg_hbm.at[pl.ds(g0, CHB // 16)], lgc)
        pltpu.sync_copy(dst_hbm.at[pl.ds(base, CHB)], didx)
        for g in range(CHB // 16):
            for h in range(H):
                tmph[h, :] = jnp.exp(lgc[g, h, :] - gsplat[h, :])
            for e in range(16):
                esplat = jnp.full((16,), e, I32)
                exbuf[g * 16 + e, :] = plsc.load_gather(tmph, [row_iota, esplat])
        pltpu.sync_copy(exbuf, ex_hbm.at[pl.ds(base, CHB)])
        pltpu.sync_copy(exbuf, densh.at[didx], add=True)

    plsc.subcore_barrier()

    @pl.when(jnp.logical_and(sid == 0, cid == 0))
    def _():
        pltpu.sync_copy(densh, den0_hbm)

    @pl.when(jnp.logical_and(sid == 0, cid == 1))
    def _():
        pltpu.sync_copy(densh, den1_hbm)


def _phase_b():
    return pl.kernel(
        _phase_b_body,
        out_type=[jax.ShapeDtypeStruct((E, 16), F32),
                  jax.ShapeDtypeStruct((N, 16), F32),
                  jax.ShapeDtypeStruct((N, 16), F32)],
        scratch_types=[
            pltpu.VMEM((NW, 16, 16), F32),
            pltpu.VMEM((16, 16), F32),
            pltpu.VMEM((CHB // 16, 16, 16), F32),
            pltpu.VMEM((16, 16), F32),
            pltpu.VMEM((CHB, 16), F32),
            pltpu.VMEM((CHB,), I32),
            pltpu.VMEM_SHARED((N, 16), F32),
        ],
        **_SC_PARAMS,
    )


def _phase_c_body(xla, xlb, ex_hbm, den0_hbm, den1_hbm, srcc, dstc,
                  outz_hbm, out0_hbm, out1_hbm,
                  xba0, xba1, xbb0, xbb1, exb0, exb1, d0b0, d0b1, d1b0, d1b1,
                  abuf, wbuf0, wbuf1, sallv, dallv, sem0, sem1, semw0, semw1,
                  outsh):
    cid, sid, wid = _worker()
    ebase = wid * EPW

    @pl.when(sid == 0)
    def _():
        pltpu.sync_copy(outz_hbm, outsh)
    plsc.subcore_barrier()

    xbas, xbbs = (xba0, xba1), (xbb0, xbb1)
    exbs, d0bs, d1bs = (exb0, exb1), (d0b0, d0b1), (d1b0, d1b1)
    sems = (sem0, sem1)
    wbufs, semw = (wbuf0, wbuf1), (semw0, semw1)

    def issue(blk, ci, b):
        base = ebase + (blk * CBLK + ci) * CHC
        pltpu.async_copy(xla.at[sallv.at[ci]], xbas[b], sems[b])
        pltpu.async_copy(xlb.at[sallv.at[ci]], xbbs[b], sems[b])
        pltpu.async_copy(ex_hbm.at[pl.ds(base, CHC)], exbs[b], sems[b])
        pltpu.async_copy(den0_hbm.at[dallv.at[ci]], d0bs[b], sems[b])
        pltpu.async_copy(den1_hbm.at[dallv.at[ci]], d1bs[b], sems[b])

    def wait(b):
        pltpu.make_async_copy(xla.at[sallv.at[0]], xbas[b], sems[b]).wait()
        pltpu.make_async_copy(xlb.at[sallv.at[0]], xbbs[b], sems[b]).wait()
        pltpu.make_async_copy(ex_hbm.at[pl.ds(ebase, CHC)], exbs[b], sems[b]).wait()
        pltpu.make_async_copy(den0_hbm.at[dallv.at[0]], d0bs[b], sems[b]).wait()
        pltpu.make_async_copy(den1_hbm.at[dallv.at[0]], d1bs[b], sems[b]).wait()

    def compute(blk, ci, b):
        xba, xbb, exb, d0b, d1b = xbas[b], xbbs[b], exbs[b], d0bs[b], d1bs[b]
        wbuf = wbufs[b]
        ilv = plsc.PackFormat.INTERLEAVED

        @pl.when(blk * CBLK + ci >= 2)
        def _():
            pltpu.make_async_copy(wbuf, outsh.at[dallv.at[ci]], semw[b]).wait()

        @pl.loop(0, CHC)
        def _alpha(e):
            d = (d0b[e, :] + d1b[e, :]) * float(H)
            abuf[e, :] = exb[e, :] / d

        @pl.loop(0, CHC)
        def _edge(e):
            zeros8 = tuple(jnp.zeros((16,), F32) for _ in range(8))
            esplat = jnp.full((16,), e, I32)

            @pl.loop(0, 8, init_carry=zeros8)
            def _lo(hh, accs):
                alv = plsc.load_gather(abuf, [esplat, jnp.full((16,), hh, I32)])
                out = list(accs)
                for g in range(4):
                    x0, x1 = plsc.unpack(
                        xba[e, pl.ds(hh * 128 + g * 32, 32)], format=ilv)
                    out[2 * g] = out[2 * g] + alv * x0
                    out[2 * g + 1] = out[2 * g + 1] + alv * x1
                return tuple(out)

            @pl.loop(8, 16, init_carry=_lo)
            def _hi(hh, accs):
                alv = plsc.load_gather(abuf, [esplat, jnp.full((16,), hh, I32)])
                out = list(accs)
                for g in range(4):
                    x0, x1 = plsc.unpack(
                        xbb[e, pl.ds((hh - 8) * 128 + g * 32, 32)], format=ilv)
                    out[2 * g] = out[2 * g] + alv * x0
                    out[2 * g + 1] = out[2 * g + 1] + alv * x1
                return tuple(out)

            for g in range(4):
                wbuf[e, pl.ds(g * 32, 16)] = _hi[2 * g]
                wbuf[e, pl.ds(g * 32 + 16, 16)] = _hi[2 * g + 1]
        pltpu.async_copy(wbuf, outsh.at[dallv.at[ci]], semw[b], add=True)

    for blk in range(NBLK):
        cb0 = wid * NCHC + blk * CBLK
        pltpu.sync_copy(srcc.at[pl.ds(cb0, CBLK)], sallv)
        pltpu.sync_copy(dstc.at[pl.ds(cb0, CBLK)], dallv)
        issue(blk, 0, 0)

        @pl.loop(0, CBLK // 2)
        def _pair(p):
            i0 = p * 2
            issue(blk, i0 + 1, 1)
            wait(0)
            compute(blk, i0, 0)

            @pl.when(i0 + 2 < CBLK)
            def _():
                issue(blk, i0 + 2, 0)
            wait(1)
            compute(blk, i0 + 1, 1)

    pltpu.make_async_copy(wbufs[0], outsh.at[dallv.at[0]], semw[0]).wait()
    pltpu.make_async_copy(wbufs[1], outsh.at[dallv.at[0]], semw[1]).wait()
    plsc.subcore_barrier()

    @pl.when(jnp.logical_and(sid == 0, cid == 0))
    def _():
        pltpu.sync_copy(outsh, out0_hbm)

    @pl.when(jnp.logical_and(sid == 0, cid == 1))
    def _():
        pltpu.sync_copy(outsh, out1_hbm)


def _phase_c():
    return pl.kernel(
        _phase_c_body,
        out_type=[jax.ShapeDtypeStruct((N, C), F32),
                  jax.ShapeDtypeStruct((N, C), F32)],
        scratch_types=[
            pltpu.VMEM((CHC, HHC), BF16),
            pltpu.VMEM((CHC, HHC), BF16),
            pltpu.VMEM((CHC, HHC), BF16),
            pltpu.VMEM((CHC, HHC), BF16),
            pltpu.VMEM((CHC, 16), F32),
            pltpu.VMEM((CHC, 16), F32),
            pltpu.VMEM((CHC, 16), F32),
            pltpu.VMEM((CHC, 16), F32),
            pltpu.VMEM((CHC, 16), F32),
            pltpu.VMEM((CHC, 16), F32),
            pltpu.VMEM((CHC, 16), F32),
            pltpu.VMEM((CHC, C), F32),
            pltpu.VMEM((CHC, C), F32),
            pltpu.VMEM((CBLK, CHC), I32),
            pltpu.VMEM((CBLK, CHC), I32),
            pltpu.SemaphoreType.DMA,
            pltpu.SemaphoreType.DMA,
            pltpu.SemaphoreType.DMA,
            pltpu.SemaphoreType.DMA,
            pltpu.VMEM_SHARED((N, C), F32),
        ],
        **_SC_PARAMS,
    )


def _prep_body(x_ref, wl_ref, wr_ref, pa_ref, pb_ref, qa_ref, qb_ref):
    xv = x_ref[...]
    xl = jnp.dot(xv, wl_ref[...], preferred_element_type=F32)
    xr = jnp.dot(xv, wr_ref[...], preferred_element_type=F32)
    pa_ref[...] = xl[:, :HHC].astype(BF16)
    pb_ref[...] = xl[:, HHC:].astype(BF16)
    qa_ref[...] = xr[:, :HHC].astype(BF16)
    qb_ref[...] = xr[:, HHC:].astype(BF16)


_prep = pl.pallas_call(
    _prep_body,
    grid=(25,),
    in_specs=[
        pl.BlockSpec((N // 25, C), lambda i: (i, 0)),
        pl.BlockSpec((C, HC), lambda i: (0, 0)),
        pl.BlockSpec((C, HC), lambda i: (0, 0)),
    ],
    out_specs=[pl.BlockSpec((N // 25, HHC), lambda i: (i, 0))] * 4,
    out_shape=[jax.ShapeDtypeStruct((N, HHC), BF16)] * 4,
)


def _combine_body(a_ref, b_ref, bias_ref, o_ref):
    s = a_ref[...] + b_ref[...] + bias_ref[...]
    o_ref[...] = jnp.maximum(s, 0.01 * s)


_combine = pl.pallas_call(
    _combine_body,
    grid=(10,),
    in_specs=[
        pl.BlockSpec((N // 10, C), lambda i: (i, 0)),
        pl.BlockSpec((N // 10, C), lambda i: (i, 0)),
        pl.BlockSpec((1, C), lambda i: (0, 0)),
    ],
    out_specs=pl.BlockSpec((N // 10, C), lambda i: (i, 0)),
    out_shape=jax.ShapeDtypeStruct((N, C), F32),
)


def _pool_body(h_ref, batch_ref, wc_ref, bc_ref, o_ref):
    hv = h_ref[...]
    bt = batch_ref[...]
    gids = lax.broadcasted_iota(I32, (N, NGRAPH), 1)
    oh = (bt == gids).astype(F32)
    sums = lax.dot_general(oh, hv, (((0,), (0,)), ((), ())),
                           preferred_element_type=F32)
    counts = jnp.sum(oh, axis=0)
    pooled = sums / jnp.maximum(counts, 1.0)[:, None]
    o_ref[...] = jnp.dot(pooled, wc_ref[...], preferred_element_type=F32) + bc_ref[...]


_pool = pl.pallas_call(
    _pool_body,
    out_shape=jax.ShapeDtypeStruct((NGRAPH, NCLASS), F32),
)


_PI = np.arange(C).reshape(C // 32, 16, 2).transpose(0, 2, 1).reshape(C)


def _gat_layer(h, edges, Wl, Wr, att, b_perm, denz, outz):
    srcg, dstg, srcc, dstc, dst = edges
    pa16, pb16, qa16, qb16 = _prep(h, Wl, Wr)
    att16 = att.reshape(HC).astype(BF16)
    lg, pm = _phase_a()(pa16, pb16, qa16, qb16, att16, srcg, dstg)
    ex, d0, d1 = _phase_b()(lg, pm, dst, denz)
    o0, o1 = _phase_c()(pa16, pb16, ex, d0, d1, srcc, dstc, outz)
    return _combine(o0, o1, b_perm.reshape(1, C))


def kernel(x, edge_index, batch, Wl1, Wr1, att1, b1, Wl2, Wr2, att2, b2, Wc, bc):
    src = edge_index[0]
    dst = edge_index[1]
    edges = (src.reshape(E // 16, 16), dst.reshape(E // 16, 16),
             src.reshape(E // CHC, CHC), dst.reshape(E // CHC, CHC), dst)
    denz = jnp.zeros((N, 16), F32)
    outz = jnp.zeros((N, C), F32)
    h = _gat_layer(x, edges, Wl1, Wr1, att1, b1[_PI], denz, outz)
    h = _gat_layer(h, edges, Wl2[_PI, :], Wr2[_PI, :], att2, b2[_PI], denz, outz)
    return _pool(h, batch.reshape(N, 1).astype(I32), Wc[_PI, :],
                 bc.reshape(1, NCLASS))
